# baseline (speedup 1.0000x reference)
import jax
import jax.numpy as jnp
from jax.experimental import pallas as pl
from jax.experimental.pallas import tpu as pltpu

_TM = 2048
_MC = 256
_NS = 4


def _body(x_ref, w1_ref, w2_ref, b2_ref, w3_ref, w4_ref, b4_ref, o_ref):
    f32 = jnp.float32
    bf16 = jnp.bfloat16
    tm = x_ref.shape[0] // 2
    mid = w1_ref.shape[1]

    r = tm // _NS
    xs = [x_ref[2 * k * r:2 * (k + 1) * r] for k in range(_NS)]
    acc2 = [None] * _NS
    us = [None] * _NS
    acc4 = [None] * _NS

    for phase in range(_NS + 1):
        a = phase
        b = phase - 1
        if a < _NS:
            acc2[a] = b2_ref[...].astype(f32)
        if b >= 0:
            o = jnp.maximum(acc2[b], 0.0)
            us[b] = jnp.concatenate([o[:r], o[r:], jnp.ones((r, 1), f32)],
                                    axis=1).astype(bf16)
            acc4[b] = b4_ref[...].astype(f32)
        for m0 in range(0, mid, _MC):
            if a < _NS:
                hm = jnp.maximum(
                    jnp.dot(xs[a], w1_ref[:, m0:m0 + _MC],
                            preferred_element_type=f32).astype(bf16), 0.0)
                acc2[a] = acc2[a] + jnp.dot(hm, w2_ref[m0:m0 + _MC, :],
                                            preferred_element_type=f32)
            if b >= 0:
                h3m = jnp.maximum(
                    jnp.dot(us[b], w3_ref[:, m0:m0 + _MC],
                            preferred_element_type=f32).astype(bf16), 0.0)
                acc4[b] = acc4[b] + jnp.dot(h3m, w4_ref[m0:m0 + _MC, :],
                                            preferred_element_type=f32)
        if b >= 0:
            o_ref[b * r:(b + 1) * r] = acc4[b]


def kernel(state, next_state, W1, b1, W2, b2, W3, b3, W4, b4):
    batch, sdim = state.shape
    mid = W1.shape[1]
    out_dim = W4.shape[1]
    f32 = jnp.float32
    bf16 = jnp.bfloat16
    grid_n = batch // _TM

    ones = jnp.ones((batch, 1), f32)
    s_aug = jnp.concatenate([state, ones], axis=1).astype(bf16)
    n_aug = jnp.concatenate([next_state, ones], axis=1).astype(bf16)
    r = _TM // _NS
    x_all = jnp.concatenate(
        [s_aug.reshape(grid_n * _NS, r, sdim + 1),
         n_aug.reshape(grid_n * _NS, r, sdim + 1)],
        axis=1).reshape(grid_n * 2 * _TM, sdim + 1)
    w1_aug = jnp.concatenate([W1, b1[None, :]], axis=0).astype(bf16)
    w3_aug = jnp.concatenate([W3, b3[None, :]], axis=0).astype(bf16)

    def rows(i):
        return (i, 0)

    def fixed(i):
        return (0, 0)

    return pl.pallas_call(
        _body,
        grid=(grid_n,),
        in_specs=[
            pl.BlockSpec((2 * _TM, sdim + 1), rows),
            pl.BlockSpec((sdim + 1, mid), fixed),
            pl.BlockSpec((mid, sdim), fixed),
            pl.BlockSpec((1, sdim), fixed),
            pl.BlockSpec((2 * sdim + 1, mid), fixed),
            pl.BlockSpec((mid, out_dim), fixed),
            pl.BlockSpec((1, out_dim), fixed),
        ],
        out_specs=pl.BlockSpec((_TM, out_dim), rows),
        out_shape=jax.ShapeDtypeStruct((batch, out_dim), f32),
        compiler_params=pltpu.CompilerParams(
            dimension_semantics=("arbitrary",),
        ),
    )(x_all, w1_aug, W2.astype(bf16), b2.reshape(1, -1), w3_aug,
      W4.astype(bf16), b4.reshape(1, -1))


# speedup vs baseline: 1.0047x; 1.0047x over previous
import jax
import jax.numpy as jnp
from jax.experimental import pallas as pl
from jax.experimental.pallas import tpu as pltpu

_TM = 2048
_MC = 256
_NS = 2


def _body(s_ref, n_ref, w1_ref, w2_ref, b2_ref, w3_ref, w4_ref, b4_ref, o_ref):
    f32 = jnp.float32
    bf16 = jnp.bfloat16
    tm = s_ref.shape[0]
    mid = w1_ref.shape[1]

    r = tm // _NS
    xs = []
    for k in range(_NS):
        xk = jnp.concatenate([s_ref[k * r:(k + 1) * r],
                              n_ref[k * r:(k + 1) * r]], axis=0).astype(bf16)
        xs.append(jnp.concatenate([xk, jnp.ones((2 * r, 1), bf16)], axis=1))

    acc2 = [None] * _NS
    us = [None] * _NS
    acc4 = [None] * _NS

    for phase in range(_NS + 1):
        a = phase
        b = phase - 1
        if a < _NS:
            acc2[a] = b2_ref[...].astype(f32)
        if b >= 0:
            o = jnp.maximum(acc2[b], 0.0)
            us[b] = jnp.concatenate([o[:r], o[r:], jnp.ones((r, 1), f32)],
                                    axis=1).astype(bf16)
            acc4[b] = b4_ref[...].astype(f32)
        for m0 in range(0, mid, _MC):
            if a < _NS:
                hm = jnp.maximum(
                    jnp.dot(xs[a], w1_ref[:, m0:m0 + _MC],
                            preferred_element_type=f32).astype(bf16), 0.0)
                acc2[a] = acc2[a] + jnp.dot(hm, w2_ref[m0:m0 + _MC, :],
                                            preferred_element_type=f32)
            if b >= 0:
                h3m = jnp.maximum(
                    jnp.dot(us[b], w3_ref[:, m0:m0 + _MC],
                            preferred_element_type=f32).astype(bf16), 0.0)
                acc4[b] = acc4[b] + jnp.dot(h3m, w4_ref[m0:m0 + _MC, :],
                                            preferred_element_type=f32)
        if b >= 0:
            o_ref[b * r:(b + 1) * r] = acc4[b]


def kernel(state, next_state, W1, b1, W2, b2, W3, b3, W4, b4):
    batch, sdim = state.shape
    mid = W1.shape[1]
    out_dim = W4.shape[1]
    f32 = jnp.float32
    bf16 = jnp.bfloat16
    grid_n = batch // _TM

    w1_aug = jnp.concatenate([W1, b1[None, :]], axis=0).astype(bf16)
    w3_aug = jnp.concatenate([W3, b3[None, :]], axis=0).astype(bf16)

    def rows(i):
        return (i, 0)

    def fixed(i):
        return (0, 0)

    return pl.pallas_call(
        _body,
        grid=(grid_n,),
        in_specs=[
            pl.BlockSpec((_TM, sdim), rows),
            pl.BlockSpec((_TM, sdim), rows),
            pl.BlockSpec((sdim + 1, mid), fixed),
            pl.BlockSpec((mid, sdim), fixed),
            pl.BlockSpec((1, sdim), fixed),
            pl.BlockSpec((2 * sdim + 1, mid), fixed),
            pl.BlockSpec((mid, out_dim), fixed),
            pl.BlockSpec((1, out_dim), fixed),
        ],
        out_specs=pl.BlockSpec((_TM, out_dim), rows),
        out_shape=jax.ShapeDtypeStruct((batch, out_dim), f32),
        compiler_params=pltpu.CompilerParams(
            dimension_semantics=("arbitrary",),
        ),
    )(state, next_state, w1_aug, W2.astype(bf16), b2.reshape(1, -1), w3_aug,
      W4.astype(bf16), b4.reshape(1, -1))


# speedup vs baseline: 1.0214x; 1.0166x over previous
import jax
import jax.numpy as jnp
from jax.experimental import pallas as pl
from jax.experimental.pallas import tpu as pltpu

_TM = 2048
_MC = 256
_NS = 2


def _body(s_ref, n_ref, w1_ref, b1_ref, w2_ref, b2_ref, w3_ref, b3_ref,
          w4_ref, b4_ref, o_ref, w1s, w2s, w3s, w4s):
    f32 = jnp.float32
    bf16 = jnp.bfloat16
    tm = s_ref.shape[0]
    mid = w1_ref.shape[1]

    @pl.when(pl.program_id(0) == 0)
    def _stage():
        w1s[:-1, :] = w1_ref[...].astype(bf16)
        w1s[-1:, :] = b1_ref[...].astype(bf16)
        w2s[...] = w2_ref[...].astype(bf16)
        w3s[:-1, :] = w3_ref[...].astype(bf16)
        w3s[-1:, :] = b3_ref[...].astype(bf16)
        w4s[...] = w4_ref[...].astype(bf16)

    r = tm // _NS
    xs = []
    for k in range(_NS):
        xk = jnp.concatenate([s_ref[k * r:(k + 1) * r],
                              n_ref[k * r:(k + 1) * r]], axis=0).astype(bf16)
        xs.append(jnp.concatenate([xk, jnp.ones((2 * r, 1), bf16)], axis=1))

    acc2 = [None] * _NS
    us = [None] * _NS
    acc4 = [None] * _NS

    for phase in range(_NS + 1):
        a = phase
        b = phase - 1
        if a < _NS:
            acc2[a] = b2_ref[...].astype(f32)
        if b >= 0:
            o = jnp.maximum(acc2[b], 0.0)
            us[b] = jnp.concatenate([o[:r], o[r:], jnp.ones((r, 1), f32)],
                                    axis=1).astype(bf16)
            acc4[b] = b4_ref[...].astype(f32)
        for m0 in range(0, mid, _MC):
            if a < _NS:
                hm = jnp.maximum(
                    jnp.dot(xs[a], w1s[:, m0:m0 + _MC],
                            preferred_element_type=f32).astype(bf16), 0.0)
                acc2[a] = acc2[a] + jnp.dot(hm, w2s[m0:m0 + _MC, :],
                                            preferred_element_type=f32)
            if b >= 0:
                h3m = jnp.maximum(
                    jnp.dot(us[b], w3s[:, m0:m0 + _MC],
                            preferred_element_type=f32).astype(bf16), 0.0)
                acc4[b] = acc4[b] + jnp.dot(h3m, w4s[m0:m0 + _MC, :],
                                            preferred_element_type=f32)
        if b >= 0:
            o_ref[b * r:(b + 1) * r] = acc4[b]


def kernel(state, next_state, W1, b1, W2, b2, W3, b3, W4, b4):
    batch, sdim = state.shape
    mid = W1.shape[1]
    out_dim = W4.shape[1]
    f32 = jnp.float32
    bf16 = jnp.bfloat16
    grid_n = batch // _TM

    def rows(i):
        return (i, 0)

    def fixed(i):
        return (0, 0)

    return pl.pallas_call(
        _body,
        grid=(grid_n,),
        in_specs=[
            pl.BlockSpec((_TM, sdim), rows),
            pl.BlockSpec((_TM, sdim), rows),
            pl.BlockSpec((sdim, mid), fixed),
            pl.BlockSpec((1, mid), fixed),
            pl.BlockSpec((mid, sdim), fixed),
            pl.BlockSpec((1, sdim), fixed),
            pl.BlockSpec((2 * sdim, mid), fixed),
            pl.BlockSpec((1, mid), fixed),
            pl.BlockSpec((mid, out_dim), fixed),
            pl.BlockSpec((1, out_dim), fixed),
        ],
        out_specs=pl.BlockSpec((_TM, out_dim), rows),
        out_shape=jax.ShapeDtypeStruct((batch, out_dim), f32),
        scratch_shapes=[
            pltpu.VMEM((sdim + 1, mid), bf16),
            pltpu.VMEM((mid, sdim), bf16),
            pltpu.VMEM((2 * sdim + 1, mid), bf16),
            pltpu.VMEM((mid, out_dim), bf16),
        ],
        compiler_params=pltpu.CompilerParams(
            dimension_semantics=("arbitrary",),
        ),
    )(state, next_state, W1, b1.reshape(1, -1), W2, b2.reshape(1, -1),
      W3, b3.reshape(1, -1), W4, b4.reshape(1, -1))
